# baseline (speedup 1.0000x reference)
import functools

import jax
import jax.numpy as jnp
from jax import lax
from jax.experimental import pallas as pl
from jax.experimental.pallas import tpu as pltpu
from jax.experimental.pallas import tpu_sc as plsc

N_NODES = 100000
N_EDGES = 1600000
D = 128
T = 128
E_FEAT = 16
B = 8192
K = 20

NC = 2
NS = 16
NW = NC * NS

CHUNK = 128
BK = B * K
NBR_PER_W = BK // NW
SRC_PER_W = B // NW
NBR_CHUNKS = NBR_PER_W // CHUNK
SRC_CHUNKS = SRC_PER_W // CHUNK

BT = 256
GRID = B // BT


def _pipelined_gather(table_hbm, idx_v, out_hbm, out_base, n_chunks, row_w,
                      buf0, buf1, sem0, sem1):
    def issue(c, buf, sem):
        pltpu.async_copy(table_hbm.at[idx_v.at[pl.ds(c * CHUNK, CHUNK)]],
                         buf, sem)

    def drain(buf, sem):
        pltpu.make_async_copy(table_hbm.at[pl.ds(0, CHUNK)], buf, sem).wait()

    issue(0, buf0, sem0)

    def body(j, carry):
        c = 2 * j
        issue(c + 1, buf1, sem1)
        drain(buf0, sem0)
        pltpu.sync_copy(buf0, out_hbm.at[pl.ds(out_base + c * CHUNK, CHUNK)])

        @pl.when(c + 2 < n_chunks)
        def _():
            issue(c + 2, buf0, sem0)

        drain(buf1, sem1)
        pltpu.sync_copy(buf1,
                        out_hbm.at[pl.ds(out_base + (c + 1) * CHUNK, CHUNK)])
        return carry

    lax.fori_loop(0, n_chunks // 2, body, 0)


def _sc_gather(memory, etp, nbr_idx, efp_idx, src_idx):
    @functools.partial(
        pl.kernel,
        mesh=plsc.VectorSubcoreMesh(core_axis_name="c", subcore_axis_name="s"),
        out_type=(
            jax.ShapeDtypeStruct((BK, D), jnp.float32),
            jax.ShapeDtypeStruct((BK, D), jnp.float32),
            jax.ShapeDtypeStruct((B, D), jnp.float32),
        ),
        scratch_types=[
            pltpu.VMEM((NBR_PER_W,), jnp.int32),
            pltpu.VMEM((NBR_PER_W,), jnp.int32),
            pltpu.VMEM((SRC_PER_W,), jnp.int32),
            pltpu.VMEM((CHUNK, D), jnp.float32),
            pltpu.VMEM((CHUNK, D), jnp.float32),
            pltpu.SemaphoreType.DMA,
            pltpu.SemaphoreType.DMA,
        ],
    )
    def gather_kernel(mem_hbm, etp_hbm, nbr_idx_hbm, efp_idx_hbm, src_idx_hbm,
                      nbr_out, efw_out, src_out,
                      nbr_idx_v, ef_idx_v, src_idx_v, buf0, buf1, sem0, sem1):
        wid = lax.axis_index("s") * NC + lax.axis_index("c")
        nbase = wid * NBR_PER_W
        sbase = wid * SRC_PER_W
        pltpu.sync_copy(nbr_idx_hbm.at[pl.ds(nbase, NBR_PER_W)], nbr_idx_v)
        pltpu.sync_copy(efp_idx_hbm.at[pl.ds(nbase, NBR_PER_W)], ef_idx_v)
        pltpu.sync_copy(src_idx_hbm.at[pl.ds(sbase, SRC_PER_W)], src_idx_v)
        _pipelined_gather(mem_hbm, nbr_idx_v, nbr_out, nbase, NBR_CHUNKS, D,
                          buf0, buf1, sem0, sem1)
        _pipelined_gather(etp_hbm, ef_idx_v, efw_out, nbase, NBR_CHUNKS, D,
                          buf0, buf1, sem0, sem1)
        _pipelined_gather(mem_hbm, src_idx_v, src_out, sbase, SRC_CHUNKS, D,
                          buf0, buf1, sem0, sem1)

    return gather_kernel(memory, etp, nbr_idx, efp_idx, src_idx)


_INV_2PI = 0.15915494309189535
_TWO_PI = 6.283185307179586
_COS_C = (9.99999989e-01, -4.99999891e-01, 4.16664892e-02, -1.38878036e-03,
          2.47698835e-05, -2.70790307e-07, 1.72450915e-09)


def _fast_cos(x):
    n = jnp.floor(x * _INV_2PI + 0.5)
    r = x - n * _TWO_PI
    u = r * r
    acc = jnp.float32(_COS_C[-1])
    for c in _COS_C[-2::-1]:
        acc = acc * u + jnp.float32(c)
    return acc


def _tc_body(nbr_ref, efw_ref, src_ref, dt_ref, id_ref, eid_ref, wt_ref,
             bt_ref, W1ab_ref, W1cb_ref, b1_ref, W2_ref, b2_ref, Ws1_ref,
             bs1_ref, Ws2_ref, bs2_ref, Wc_ref, bc_ref, out_ref):
    dt = dt_ref[...]
    m_all = (id_ref[...] != 0).astype(jnp.float32)
    emod = jnp.bitwise_and(eid_ref[...] >> 3, 7)
    lane_grp = jax.lax.broadcasted_iota(jnp.int32, (BT, D), 1) >> 4
    wt = wt_ref[...]
    bt = bt_ref[...]
    W1ab = W1ab_ref[...]
    W1cb = W1cb_ref[...]
    b1 = b1_ref[...]
    acc = jnp.zeros((BT, D), jnp.float32)
    for k in range(K):
        nbr_k = nbr_ref[k]
        efw_k = efw_ref[k]
        d_col = lax.slice(dt, (0, k), (BT, k + 1))
        ete = _fast_cos(d_col * wt + bt)
        x = jnp.concatenate([nbr_k, ete], axis=1)
        mod_col = lax.slice(emod, (0, k), (BT, k + 1))
        zw = jnp.where(lane_grp == mod_col, efw_k, 0.0)
        pre = (jnp.dot(x, W1ab, preferred_element_type=jnp.float32)
               + jnp.dot(zw, W1cb, preferred_element_type=jnp.float32)
               + b1)
        h = jnp.maximum(pre, 0.0)
        m = lax.slice(m_all, (0, k), (BT, k + 1))
        acc = acc + h * m
    cnt = jnp.sum(m_all, axis=1, keepdims=True)
    agg = (jnp.dot(acc, W2_ref[...], preferred_element_type=jnp.float32)
           + cnt * b2_ref[...])
    s = jnp.maximum(
        jnp.dot(src_ref[...], Ws1_ref[...], preferred_element_type=jnp.float32)
        + bs1_ref[...], 0.0)
    s = jnp.dot(s, Ws2_ref[...], preferred_element_type=jnp.float32) + bs2_ref[...]
    out_ref[...] = (jnp.dot(agg, Wc_ref[0:D, :], preferred_element_type=jnp.float32)
                    + jnp.dot(s, Wc_ref[D:2 * D, :], preferred_element_type=jnp.float32)
                    + bc_ref[...])


def _tc_dense(nbrg, efwg, srcg, deltas, ids, eids, wt2, bt2, W1ab, W1cb, b12,
              W2, b22, Ws1, bs12, Ws2, bs22, Wc, bc2):
    full = lambda shape: pl.BlockSpec(shape, lambda i: tuple(0 for _ in shape))
    return pl.pallas_call(
        _tc_body,
        grid=(GRID,),
        in_specs=[
            pl.BlockSpec((K, BT, D), lambda i: (0, i, 0)),
            pl.BlockSpec((K, BT, D), lambda i: (0, i, 0)),
            pl.BlockSpec((BT, D), lambda i: (i, 0)),
            pl.BlockSpec((BT, K), lambda i: (i, 0)),
            pl.BlockSpec((BT, K), lambda i: (i, 0)),
            pl.BlockSpec((BT, K), lambda i: (i, 0)),
            full((1, T)),
            full((1, T)),
            full((D + T, D)),
            full((D, D)),
            full((1, D)),
            full((D, D)),
            full((1, D)),
            full((D, D)),
            full((1, D)),
            full((D, D)),
            full((1, D)),
            full((2 * D, D)),
            full((1, D)),
        ],
        out_specs=pl.BlockSpec((BT, D), lambda i: (i, 0)),
        out_shape=jax.ShapeDtypeStruct((B, D), jnp.float32),
    )(nbrg, efwg, srcg, deltas, ids, eids, wt2, bt2, W1ab, W1cb, b12,
      W2, b22, Ws1, bs12, Ws2, bs22, Wc, bc2)


def kernel(memory, edge_table, edge_deltas, w_t, b_t, W1, b1, W2, b2,
           Ws1, bs1, Ws2, bs2, Wc, bc, source_nodes, neighbors, edge_idxs):
    nbr_idx = neighbors.T.reshape(-1).astype(jnp.int32)
    ef_t = edge_idxs.T.reshape(-1)
    efp_idx = (((ef_t >> 6) << 3) | (ef_t & 7)).astype(jnp.int32)
    src_idx = source_nodes.astype(jnp.int32)
    etp = (edge_table.reshape(N_EDGES // 64, 8, 8, E_FEAT)
           .transpose(0, 2, 1, 3).reshape(N_EDGES // 8, 8 * E_FEAT))
    nbrg, efwg, srcg = _sc_gather(memory, etp, nbr_idx, efp_idx, src_idx)
    nbrg = nbrg.reshape(K, B, D)
    efwg = efwg.reshape(K, B, D)
    W1ab = W1[:D + T]
    W1cb = jnp.tile(W1[D + T:], (8, 1))
    return _tc_dense(
        nbrg, efwg, srcg, edge_deltas, neighbors, edge_idxs,
        w_t.reshape(1, T), b_t.reshape(1, T), W1ab, W1cb, b1.reshape(1, D),
        W2, b2.reshape(1, D), Ws1, bs1.reshape(1, D), Ws2, bs2.reshape(1, D),
        Wc, bc.reshape(1, D))


# speedup vs baseline: 1.3489x; 1.3489x over previous
import functools

import jax
import jax.numpy as jnp
from jax import lax
from jax.experimental import pallas as pl
from jax.experimental.pallas import tpu as pltpu
from jax.experimental.pallas import tpu_sc as plsc

N_NODES = 100000
N_EDGES = 1600000
D = 128
T = 128
E_FEAT = 16
B = 8192
K = 20

NC = 2
NS = 16
NW = NC * NS

CHUNK = 128
BK = B * K
NBR_PER_W = BK // NW
SRC_PER_W = B // NW
NBR_CHUNKS = NBR_PER_W // CHUNK
SRC_CHUNKS = SRC_PER_W // CHUNK

BT = 256
GRID = B // BT


def _pipelined_gather(table_hbm, idx_v, out_hbm, out_base, n_chunks, row_w,
                      buf0, buf1, sem0, sem1):
    def issue(c, buf, sem):
        pltpu.async_copy(table_hbm.at[idx_v.at[pl.ds(c * CHUNK, CHUNK)]],
                         buf, sem)

    def drain(buf, sem):
        pltpu.make_async_copy(table_hbm.at[pl.ds(0, CHUNK)], buf, sem).wait()

    issue(0, buf0, sem0)

    def body(j, carry):
        c = 2 * j
        issue(c + 1, buf1, sem1)
        drain(buf0, sem0)
        pltpu.sync_copy(buf0, out_hbm.at[pl.ds(out_base + c * CHUNK, CHUNK)])

        @pl.when(c + 2 < n_chunks)
        def _():
            issue(c + 2, buf0, sem0)

        drain(buf1, sem1)
        pltpu.sync_copy(buf1,
                        out_hbm.at[pl.ds(out_base + (c + 1) * CHUNK, CHUNK)])
        return carry

    lax.fori_loop(0, n_chunks // 2, body, 0)


def _sc_gather_mem(memory, nbr_idx, src_idx):
    @functools.partial(
        pl.kernel,
        mesh=plsc.VectorSubcoreMesh(core_axis_name="c", subcore_axis_name="s"),
        out_type=(
            jax.ShapeDtypeStruct((BK, D), jnp.float32),
            jax.ShapeDtypeStruct((B, D), jnp.float32),
        ),
        scratch_types=[
            pltpu.VMEM((NBR_PER_W,), jnp.int32),
            pltpu.VMEM((SRC_PER_W,), jnp.int32),
            pltpu.VMEM((CHUNK, D), jnp.float32),
            pltpu.VMEM((CHUNK, D), jnp.float32),
            pltpu.SemaphoreType.DMA,
            pltpu.SemaphoreType.DMA,
        ],
    )
    def gather_kernel(mem_hbm, nbr_idx_hbm, src_idx_hbm,
                      nbr_out, src_out,
                      nbr_idx_v, src_idx_v, buf0, buf1, sem0, sem1):
        wid = lax.axis_index("s") * NC + lax.axis_index("c")
        nbase = wid * NBR_PER_W
        sbase = wid * SRC_PER_W
        pltpu.sync_copy(nbr_idx_hbm.at[pl.ds(nbase, NBR_PER_W)], nbr_idx_v)
        pltpu.sync_copy(src_idx_hbm.at[pl.ds(sbase, SRC_PER_W)], src_idx_v)
        _pipelined_gather(mem_hbm, nbr_idx_v, nbr_out, nbase, NBR_CHUNKS, D,
                          buf0, buf1, sem0, sem1)
        _pipelined_gather(mem_hbm, src_idx_v, src_out, sbase, SRC_CHUNKS, D,
                          buf0, buf1, sem0, sem1)

    return gather_kernel(memory, nbr_idx, src_idx)


def _sc_gather_ef(etp, efp_idx):
    @functools.partial(
        pl.kernel,
        mesh=plsc.VectorSubcoreMesh(core_axis_name="c", subcore_axis_name="s"),
        out_type=jax.ShapeDtypeStruct((BK, D), jnp.float32),
        scratch_types=[
            pltpu.VMEM((NBR_PER_W,), jnp.int32),
            pltpu.VMEM((CHUNK, D), jnp.float32),
            pltpu.VMEM((CHUNK, D), jnp.float32),
            pltpu.SemaphoreType.DMA,
            pltpu.SemaphoreType.DMA,
        ],
    )
    def gather_kernel(etp_hbm, efp_idx_hbm, efw_out,
                      ef_idx_v, buf0, buf1, sem0, sem1):
        wid = lax.axis_index("s") * NC + lax.axis_index("c")
        nbase = wid * NBR_PER_W
        pltpu.sync_copy(efp_idx_hbm.at[pl.ds(nbase, NBR_PER_W)], ef_idx_v)
        _pipelined_gather(etp_hbm, ef_idx_v, efw_out, nbase, NBR_CHUNKS, D,
                          buf0, buf1, sem0, sem1)

    return gather_kernel(etp, efp_idx)


_INV_2PI = 0.15915494309189535
_TWO_PI = 6.283185307179586
_COS_C = (9.99999989e-01, -4.99999891e-01, 4.16664892e-02, -1.38878036e-03,
          2.47698835e-05, -2.70790307e-07, 1.72450915e-09)


def _fast_cos(x):
    n = jnp.floor(x * _INV_2PI + 0.5)
    r = x - n * _TWO_PI
    u = r * r
    acc = jnp.float32(_COS_C[-1])
    for c in _COS_C[-2::-1]:
        acc = acc * u + jnp.float32(c)
    return acc


def _tc_body(nbr_ref, efw_ref, src_ref, dt_ref, id_ref, eid_ref, wt_ref,
             bt_ref, W1ab_ref, W1cb_ref, b1_ref, W2_ref, b2_ref, Ws1_ref,
             bs1_ref, Ws2_ref, bs2_ref, Wc_ref, bc_ref, out_ref):
    dt = dt_ref[...]
    m_all = (id_ref[...] != 0).astype(jnp.float32)
    emod = jnp.bitwise_and(eid_ref[...], 7)
    lane_grp = jax.lax.broadcasted_iota(jnp.int32, (BT, D), 1) >> 4
    wt = wt_ref[...]
    bt = bt_ref[...]
    W1ab = W1ab_ref[...]
    W1cb = W1cb_ref[...]
    b1 = b1_ref[...]
    acc = jnp.zeros((BT, D), jnp.float32)
    for k in range(K):
        nbr_k = nbr_ref[k]
        efw_k = efw_ref[k]
        d_col = lax.slice(dt, (0, k), (BT, k + 1))
        ete = _fast_cos(d_col * wt + bt)
        x = jnp.concatenate([nbr_k, ete], axis=1)
        mod_col = lax.slice(emod, (0, k), (BT, k + 1))
        zw = jnp.where(lane_grp == mod_col, efw_k, 0.0)
        pre = (jnp.dot(x, W1ab, preferred_element_type=jnp.float32)
               + jnp.dot(zw, W1cb, preferred_element_type=jnp.float32)
               + b1)
        h = jnp.maximum(pre, 0.0)
        m = lax.slice(m_all, (0, k), (BT, k + 1))
        acc = acc + h * m
    cnt = jnp.sum(m_all, axis=1, keepdims=True)
    agg = (jnp.dot(acc, W2_ref[...], preferred_element_type=jnp.float32)
           + cnt * b2_ref[...])
    s = jnp.maximum(
        jnp.dot(src_ref[...], Ws1_ref[...], preferred_element_type=jnp.float32)
        + bs1_ref[...], 0.0)
    s = jnp.dot(s, Ws2_ref[...], preferred_element_type=jnp.float32) + bs2_ref[...]
    out_ref[...] = (jnp.dot(agg, Wc_ref[0:D, :], preferred_element_type=jnp.float32)
                    + jnp.dot(s, Wc_ref[D:2 * D, :], preferred_element_type=jnp.float32)
                    + bc_ref[...])


def _tc_dense(nbrg, efwg, srcg, deltas, ids, eids, wt2, bt2, W1ab, W1cb, b12,
              W2, b22, Ws1, bs12, Ws2, bs22, Wc, bc2):
    full = lambda shape: pl.BlockSpec(shape, lambda i: tuple(0 for _ in shape))
    return pl.pallas_call(
        _tc_body,
        grid=(GRID,),
        in_specs=[
            pl.BlockSpec((K, BT, D), lambda i: (0, i, 0)),
            pl.BlockSpec((K, BT, D), lambda i: (0, i, 0)),
            pl.BlockSpec((BT, D), lambda i: (i, 0)),
            pl.BlockSpec((BT, K), lambda i: (i, 0)),
            pl.BlockSpec((BT, K), lambda i: (i, 0)),
            pl.BlockSpec((BT, K), lambda i: (i, 0)),
            full((1, T)),
            full((1, T)),
            full((D + T, D)),
            full((D, D)),
            full((1, D)),
            full((D, D)),
            full((1, D)),
            full((D, D)),
            full((1, D)),
            full((D, D)),
            full((1, D)),
            full((2 * D, D)),
            full((1, D)),
        ],
        out_specs=pl.BlockSpec((BT, D), lambda i: (i, 0)),
        out_shape=jax.ShapeDtypeStruct((B, D), jnp.float32),
    )(nbrg, efwg, srcg, deltas, ids, eids, wt2, bt2, W1ab, W1cb, b12,
      W2, b22, Ws1, bs12, Ws2, bs22, Wc, bc2)


def kernel(memory, edge_table, edge_deltas, w_t, b_t, W1, b1, W2, b2,
           Ws1, bs1, Ws2, bs2, Wc, bc, source_nodes, neighbors, edge_idxs):
    nbr_idx = neighbors.T.reshape(-1).astype(jnp.int32)
    efp_idx = (edge_idxs.T.reshape(-1) >> 3).astype(jnp.int32)
    src_idx = source_nodes.astype(jnp.int32)
    etp = edge_table.reshape(N_EDGES // 8, 8 * E_FEAT)
    nbrg, srcg = _sc_gather_mem(memory, nbr_idx, src_idx)
    efwg = _sc_gather_ef(etp, efp_idx)
    nbrg = nbrg.reshape(K, B, D)
    efwg = efwg.reshape(K, B, D)
    W1ab = W1[:D + T]
    W1cb = jnp.tile(W1[D + T:], (8, 1))
    return _tc_dense(
        nbrg, efwg, srcg, edge_deltas, neighbors, edge_idxs,
        w_t.reshape(1, T), b_t.reshape(1, T), W1ab, W1cb, b1.reshape(1, D),
        W2, b2.reshape(1, D), Ws1, bs1.reshape(1, D), Ws2, bs2.reshape(1, D),
        Wc, bc.reshape(1, D))
